# baseline (speedup 1.0000x reference)
import functools

import jax
import jax.numpy as jnp
from jax import lax
from jax.experimental import pallas as pl
from jax.experimental.pallas import tpu as pltpu
from jax.experimental.pallas import tpu_sc as plsc

BATCH = 16384
DIM = 128
NC = 2
NS = 16
NW = NC * NS
B_PER_W = BATCH // NW
CHUNK = 128
NCHUNK = B_PER_W // CHUNK
LANES = 16
GROUPS = CHUNK // LANES


_GATHER_DNUMS = lax.GatherDimensionNumbers(
    offset_dims=(), collapsed_slice_dims=(0,), start_index_map=(0,))


def _shuffle(x, idx):
    return lax.gather(x, idx[:, None], _GATHER_DNUMS, slice_sizes=(1,),
                      mode=lax.GatherScatterMode.PROMISE_IN_BOUNDS)


def _sc_body(user_id, item_id, user_table, item_table, out,
             idx_u, idx_i, rows_u, rows_i, out_v, sem_u, sem_i):
    wid = lax.axis_index("s") * NC + lax.axis_index("c")
    base = wid * B_PER_W
    lane = lax.iota(jnp.int32, 16)

    for c in range(NCHUNK):
        off = base + c * CHUNK
        pltpu.sync_copy(user_id.at[pl.ds(off, CHUNK)], idx_u.at[c])
        pltpu.sync_copy(item_id.at[pl.ds(off, CHUNK)], idx_i.at[c])
        cp_u = pltpu.async_copy(user_table.at[idx_u.at[c]], rows_u, sem_u)
        cp_i = pltpu.async_copy(item_table.at[idx_i.at[c]], rows_i, sem_i)
        cp_u.wait()
        cp_i.wait()

        def group_body(g, _, c=c):
            out_vec = jnp.zeros((16,), jnp.float32)
            for j in range(LANES):
                row = g * LANES + j
                acc = jnp.zeros((16,), jnp.float32)
                for s in range(DIM // 16):
                    u = rows_u[row, pl.ds(s * 16, 16)]
                    v = rows_i[row, pl.ds(s * 16, 16)]
                    acc = acc + u * v
                for m in (8, 4, 2, 1):
                    acc = acc + _shuffle(acc, lane ^ m)
                out_vec = jnp.where(lane == j, acc, out_vec)
            out_v[pl.ds(c * CHUNK + g * LANES, 16)] = out_vec
            return 0

        lax.fori_loop(0, GROUPS, group_body, 0)

    pltpu.sync_copy(out_v, out.at[pl.ds(base, B_PER_W)])


@jax.jit
def kernel(user_id, item_id, user_table, item_table):
    mesh = plsc.VectorSubcoreMesh(
        core_axis_name="c", subcore_axis_name="s",
        num_cores=NC, num_subcores=NS)
    run = pl.kernel(
        _sc_body,
        out_type=jax.ShapeDtypeStruct((BATCH,), jnp.float32),
        mesh=mesh,
        scratch_types=[
            pltpu.VMEM((NCHUNK, CHUNK), jnp.int32),
            pltpu.VMEM((NCHUNK, CHUNK), jnp.int32),
            pltpu.VMEM((CHUNK, DIM), jnp.float32),
            pltpu.VMEM((CHUNK, DIM), jnp.float32),
            pltpu.VMEM((B_PER_W,), jnp.float32),
            pltpu.SemaphoreType.DMA,
            pltpu.SemaphoreType.DMA,
        ],
    )
    return run(user_id, item_id, user_table, item_table)


# speedup vs baseline: 1.1137x; 1.1137x over previous
import jax
import jax.numpy as jnp
from jax import lax
from jax.experimental import pallas as pl
from jax.experimental.pallas import tpu as pltpu
from jax.experimental.pallas import tpu_sc as plsc

BATCH = 16384
DIM = 128
NC = 2
NS = 16
NW = NC * NS
B_PER_W = BATCH // NW
CHUNK = 64
NCHUNK = B_PER_W // CHUNK
LANES = 16
GROUPS = CHUNK // LANES

_GATHER_DNUMS = lax.GatherDimensionNumbers(
    offset_dims=(), collapsed_slice_dims=(0,), start_index_map=(0,))


def _shuffle(x, idx):
    return lax.gather(x, idx[:, None], _GATHER_DNUMS, slice_sizes=(1,),
                      mode=lax.GatherScatterMode.PROMISE_IN_BOUNDS)


def _sc_body(user_id, item_id, user_table, item_table, out,
             idx_u, idx_i, rows_u, rows_i, out_v,
             sem_u0, sem_u1, sem_i0, sem_i1):
    sem_u = (sem_u0, sem_u1)
    sem_i = (sem_i0, sem_i1)
    wid = lax.axis_index("s") * NC + lax.axis_index("c")
    base = wid * B_PER_W
    lane = lax.iota(jnp.int32, 16)

    cp_u = pltpu.async_copy(user_id.at[pl.ds(wid * NCHUNK, NCHUNK)],
                            idx_u, sem_u0)
    cp_i = pltpu.async_copy(item_id.at[pl.ds(wid * NCHUNK, NCHUNK)],
                            idx_i, sem_i0)
    cp_u.wait()
    cp_i.wait()

    def start(cc, b):
        return (pltpu.async_copy(user_table.at[idx_u.at[cc]],
                                 rows_u.at[b], sem_u[b]),
                pltpu.async_copy(item_table.at[idx_i.at[cc]],
                                 rows_i.at[b], sem_i[b]))

    pending = {0: start(0, 0)}
    for cc in range(NCHUNK):
        b = cc % 2
        if cc + 1 < NCHUNK:
            pending[cc + 1] = start(cc + 1, (cc + 1) % 2)
        gu, gi = pending.pop(cc)
        gu.wait()
        gi.wait()

        def group_body(g, _, cc=cc, b=b):
            out_vec = jnp.zeros((16,), jnp.float32)
            for j in range(LANES):
                row = g * LANES + j
                acc = jnp.zeros((16,), jnp.float32)
                for s in range(DIM // 16):
                    u = rows_u[b, row, pl.ds(s * 16, 16)]
                    v = rows_i[b, row, pl.ds(s * 16, 16)]
                    acc = acc + u * v
                for m in (8, 4, 2, 1):
                    acc = acc + _shuffle(acc, lane ^ m)
                out_vec = jnp.where(lane == j, acc, out_vec)
            out_v[pl.ds(cc * CHUNK + g * LANES, 16)] = out_vec
            return 0

        lax.fori_loop(0, GROUPS, group_body, 0)

    pltpu.sync_copy(out_v, out.at[pl.ds(base, B_PER_W)])


@jax.jit
def kernel(user_id, item_id, user_table, item_table):
    mesh = plsc.VectorSubcoreMesh(
        core_axis_name="c", subcore_axis_name="s",
        num_cores=NC, num_subcores=NS)
    run = pl.kernel(
        _sc_body,
        out_type=jax.ShapeDtypeStruct((BATCH,), jnp.float32),
        mesh=mesh,
        scratch_types=[
            pltpu.VMEM((NCHUNK, CHUNK), jnp.int32),
            pltpu.VMEM((NCHUNK, CHUNK), jnp.int32),
            pltpu.VMEM((2, CHUNK, DIM), jnp.float32),
            pltpu.VMEM((2, CHUNK, DIM), jnp.float32),
            pltpu.VMEM((B_PER_W,), jnp.float32),
            pltpu.SemaphoreType.DMA,
            pltpu.SemaphoreType.DMA,
            pltpu.SemaphoreType.DMA,
            pltpu.SemaphoreType.DMA,
        ],
    )
    return run(user_id.reshape(BATCH // CHUNK, CHUNK),
               item_id.reshape(BATCH // CHUNK, CHUNK),
               user_table, item_table)


# speedup vs baseline: 1.1332x; 1.0175x over previous
import jax
import jax.numpy as jnp
from jax import lax
from jax.experimental import pallas as pl
from jax.experimental.pallas import tpu as pltpu
from jax.experimental.pallas import tpu_sc as plsc

BATCH = 16384
DIM = 128
NC = 2
NS = 16
NW = NC * NS
B_PER_W = BATCH // NW
CHUNK = 64
NCHUNK = B_PER_W // CHUNK
LANES = 16
GROUPS = CHUNK // LANES

_GATHER_DNUMS = lax.GatherDimensionNumbers(
    offset_dims=(), collapsed_slice_dims=(0,), start_index_map=(0,))


def _shuffle(x, idx):
    return lax.gather(x, idx[:, None], _GATHER_DNUMS, slice_sizes=(1,),
                      mode=lax.GatherScatterMode.PROMISE_IN_BOUNDS)


def _sc_body(user_id, item_id, user_table, item_table, out,
             idx_u, idx_i, rows_u, rows_i, out_v,
             sem_u0, sem_u1, sem_i0, sem_i1):
    sem_u = (sem_u0, sem_u1)
    sem_i = (sem_i0, sem_i1)
    wid = lax.axis_index("s") * NC + lax.axis_index("c")
    base = wid * B_PER_W
    lane = lax.iota(jnp.int32, 16)

    cp_u = pltpu.async_copy(user_id.at[pl.ds(base, B_PER_W)], idx_u, sem_u0)
    cp_i = pltpu.async_copy(item_id.at[pl.ds(base, B_PER_W)], idx_i, sem_i0)
    cp_u.wait()
    cp_i.wait()

    def gathers(cc, b):
        return (pltpu.make_async_copy(
                    user_table.at[idx_u.at[pl.ds(cc * CHUNK, CHUNK)]],
                    rows_u.at[b], sem_u[b]),
                pltpu.make_async_copy(
                    item_table.at[idx_i.at[pl.ds(cc * CHUNK, CHUNK)]],
                    rows_i.at[b], sem_i[b]))

    def start(cc, b):
        gu, gi = gathers(cc, b)
        gu.start()
        gi.start()

    def compute(cc, b):
        obase = cc * CHUNK

        @plsc.parallel_loop(0, GROUPS, 1, unroll=2)
        def _(g):
            out_vec = jnp.zeros((16,), jnp.float32)
            for j in range(LANES):
                row = g * LANES + j
                acc = jnp.zeros((16,), jnp.float32)
                for s in range(DIM // 16):
                    u = rows_u[b, row, pl.ds(s * 16, 16)]
                    v = rows_i[b, row, pl.ds(s * 16, 16)]
                    acc = acc + u * v
                for m in (8, 4, 2, 1):
                    acc = acc + _shuffle(acc, lane ^ m)
                out_vec = jnp.where(lane == j, acc, out_vec)
            out_v[pl.ds(obase + g * LANES, 16)] = out_vec

    start(0, 0)

    def pair_body(p, _):
        for b in (0, 1):
            cc = 2 * p + b
            nxt = cc + 1

            @pl.when(nxt < NCHUNK)
            def _():
                start(nxt, 1 - b)

            gu, gi = gathers(cc, b)
            gu.wait()
            gi.wait()
            compute(cc, b)
        return 0

    lax.fori_loop(0, NCHUNK // 2, pair_body, 0)

    pltpu.sync_copy(out_v, out.at[pl.ds(base, B_PER_W)])


@jax.jit
def kernel(user_id, item_id, user_table, item_table):
    mesh = plsc.VectorSubcoreMesh(
        core_axis_name="c", subcore_axis_name="s",
        num_cores=NC, num_subcores=NS)
    run = pl.kernel(
        _sc_body,
        out_type=jax.ShapeDtypeStruct((BATCH,), jnp.float32),
        mesh=mesh,
        scratch_types=[
            pltpu.VMEM((B_PER_W,), jnp.int32),
            pltpu.VMEM((B_PER_W,), jnp.int32),
            pltpu.VMEM((2, CHUNK, DIM), jnp.float32),
            pltpu.VMEM((2, CHUNK, DIM), jnp.float32),
            pltpu.VMEM((B_PER_W,), jnp.float32),
            pltpu.SemaphoreType.DMA,
            pltpu.SemaphoreType.DMA,
            pltpu.SemaphoreType.DMA,
            pltpu.SemaphoreType.DMA,
        ],
    )
    return run(user_id, item_id, user_table, item_table)
